# Initial kernel scaffold; baseline (speedup 1.0000x reference)
import jax
import jax.numpy as jnp
from jax.experimental import pallas as pl


def kernel(node_feats, edge_index, edge_weight, W1, b1, W2, b2):
    raise NotImplementedError("write your pallas kernel here")


# speedup vs baseline: 2.3376x; 2.3376x over previous
import jax
import jax.numpy as jnp
from jax import lax
from jax.experimental import pallas as pl
from jax.experimental.pallas import tpu as pltpu
from jax.experimental.pallas import tpu_sc as plsc

N = 10000
D = 128
NC = 2
NS = 16
LANES = 16
_GRP = 128
_NQ = D // LANES

R_P = 2560
EP = R_P * _GRP
DG_Q = R_P // NS
C_Q = R_P // (NC * NS)

NT = N + 8
NP = 10112
ZROWS = NP // NS
WROWS = 624

_sc_mesh = plsc.VectorSubcoreMesh(
    core_axis_name="c", subcore_axis_name="s", num_cores=NC, num_subcores=NS)


def _zero_fill(buf, nrows):
  def zrow(i, carry):
    for qq in range(buf.shape[1] // LANES):
      buf[i, pl.ds(qq * LANES, LANES)] = jnp.zeros((LANES,), jnp.float32)
    return carry
  lax.fori_loop(0, nrows, zrow, 0)


def _zero_table(zb, table, s):
  base = s * ZROWS
  for k in range(4):
    pltpu.sync_copy(zb, table.at[pl.ds(base + k * 128, 128)])
  pltpu.sync_copy(zb.at[pl.ds(0, ZROWS - 512)],
                  table.at[pl.ds(base + 512, ZROWS - 512)])


def _writeout(table, out_hbm, s, out_base):
  wbase = s * WROWS
  pltpu.sync_copy(table.at[pl.ds(wbase, WROWS)],
                  out_hbm.at[pl.ds(out_base + wbase, WROWS)])

  @pl.when(s == NS - 1)
  def _tail():
    tb = NS * WROWS
    pltpu.sync_copy(table.at[pl.ds(tb, N - tb)],
                    out_hbm.at[pl.ds(out_base + tb, N - tb)])


def _deg_body(sd_hbm, degp_hbm, tab_sh, ones_b, ibuf):
  c = lax.axis_index("c")
  s = lax.axis_index("s")

  _zero_fill(ones_b, _GRP)
  _zero_table(ones_b, tab_sh, s)

  def orow(i, carry):
    for qq in range(_NQ):
      ones_b[i, pl.ds(qq * LANES, LANES)] = jnp.ones((LANES,), jnp.float32)
    return carry
  lax.fori_loop(0, _GRP, orow, 0)

  base = c * R_P + s * DG_Q
  pltpu.sync_copy(sd_hbm.at[pl.ds(base, DG_Q)], ibuf)

  plsc.subcore_barrier()

  def grp(j, carry):
    pltpu.sync_copy(ones_b, tab_sh.at[ibuf.at[j]], add=True)
    return carry
  lax.fori_loop(0, DG_Q, grp, 0)

  plsc.subcore_barrier()
  _writeout(tab_sh, degp_hbm, s, c * N)


_deg_call = pl.kernel(
    _deg_body,
    out_type=jax.ShapeDtypeStruct((2 * N, D), jnp.float32),
    mesh=_sc_mesh,
    scratch_types=[
        pltpu.VMEM_SHARED((NP, D), jnp.float32),
        pltpu.VMEM((_GRP, D), jnp.float32),
        pltpu.VMEM((DG_Q, _GRP), jnp.int32),
    ],
)


def _agg_body(xt_hbm, src_hbm, dst_hbm, w_hbm, aggp_hbm,
              agg_sh, sbuf, dbuf, wbuf, rows):
  c = lax.axis_index("c")
  s = lax.axis_index("s")

  _zero_fill(rows, _GRP)
  _zero_table(rows, agg_sh, s)

  base = (c * NS + s) * C_Q
  pltpu.sync_copy(src_hbm.at[pl.ds(base, C_Q)], sbuf)
  pltpu.sync_copy(dst_hbm.at[pl.ds(base, C_Q)], dbuf)
  pltpu.sync_copy(w_hbm.at[pl.ds(base, C_Q)], wbuf)

  plsc.subcore_barrier()

  def grp(j, carry):
    pltpu.sync_copy(xt_hbm.at[sbuf.at[j]], rows)

    def edge16(g, c2):
      wv = wbuf[j, pl.ds(g * LANES, LANES)]
      for i in range(LANES):
        ws = wv[i]
        e = g * LANES + i
        for qq in range(_NQ):
          sl = pl.ds(qq * LANES, LANES)
          rows[e, sl] = rows[e, sl] * ws
      return c2
    lax.fori_loop(0, _GRP // LANES, edge16, 0)
    pltpu.sync_copy(rows, agg_sh.at[dbuf.at[j]], add=True)
    return carry
  lax.fori_loop(0, C_Q, grp, 0)

  plsc.subcore_barrier()
  _writeout(agg_sh, aggp_hbm, s, c * N)


_agg_call = pl.kernel(
    _agg_body,
    out_type=jax.ShapeDtypeStruct((2 * N, D), jnp.float32),
    mesh=_sc_mesh,
    scratch_types=[
        pltpu.VMEM_SHARED((NP, D), jnp.float32),
        pltpu.VMEM((C_Q, _GRP), jnp.int32),
        pltpu.VMEM((C_Q, _GRP), jnp.int32),
        pltpu.VMEM((C_Q, _GRP), jnp.float32),
        pltpu.VMEM((_GRP, D), jnp.float32),
    ],
)


def _prep_body(x_ref, degp_ref, xt_ref):
  outdeg = degp_ref[0:N, 0:1]
  outr = lax.rsqrt(jnp.maximum(outdeg, 1.0))
  xt_ref[0:N, :] = x_ref[...] * outr
  xt_ref[N:NT, :] = jnp.zeros((NT - N, D), jnp.float32)


_prep = pl.pallas_call(
    _prep_body,
    out_shape=jax.ShapeDtypeStruct((NT, D), jnp.float32),
)


def _mid_body(aggp_ref, degp_ref, w_ref, b_ref, out_ref):
  agg = aggp_ref[0:N, :] + aggp_ref[N:2 * N, :]
  indeg = degp_ref[N:2 * N, 0:1]
  inr = lax.rsqrt(jnp.maximum(indeg, 1.0))
  rst = agg * inr
  h = jnp.dot(rst, w_ref[...], preferred_element_type=jnp.float32) + b_ref[...]
  h = jnp.maximum(h, 0.0)
  outdeg = degp_ref[0:N, 0:1]
  outr = lax.rsqrt(jnp.maximum(outdeg, 1.0))
  out_ref[0:N, :] = h * outr
  out_ref[N:NT, :] = jnp.zeros((NT - N, D), jnp.float32)


_mid = pl.pallas_call(
    _mid_body,
    out_shape=jax.ShapeDtypeStruct((NT, D), jnp.float32),
)


def _fin_body(aggp_ref, degp_ref, w_ref, b_ref, out_ref):
  agg = aggp_ref[0:N, :] + aggp_ref[N:2 * N, :]
  indeg = degp_ref[N:2 * N, 0:1]
  inr = lax.rsqrt(jnp.maximum(indeg, 1.0))
  rst = agg * inr
  out_ref[...] = (
      jnp.dot(rst, w_ref[...], preferred_element_type=jnp.float32) + b_ref[...])


_fin = pl.pallas_call(
    _fin_body,
    out_shape=jax.ShapeDtypeStruct((N, D), jnp.float32),
)


@jax.jit
def kernel(node_feats, edge_index, edge_weight, W1, b1, W2, b2):
  E = edge_index.shape[1]
  pad = EP - E
  src = jnp.concatenate(
      [edge_index[0], jnp.full((pad,), N, jnp.int32)]).reshape(R_P, _GRP)
  dst = jnp.concatenate(
      [edge_index[1], jnp.full((pad,), N, jnp.int32)]).reshape(R_P, _GRP)
  w2 = jnp.concatenate(
      [edge_weight, jnp.zeros((pad,), jnp.float32)]).reshape(R_P, _GRP)
  degp = _deg_call(jnp.concatenate([src, dst], axis=0))
  xt = _prep(node_feats, degp)
  agg1 = _agg_call(xt, src, dst, w2)
  hh = _mid(agg1, degp, W1, b1.reshape(1, D))
  agg2 = _agg_call(hh, src, dst, w2)
  return _fin(agg2, degp, W2, b2.reshape(1, D))


# speedup vs baseline: 5.2825x; 2.2597x over previous
import jax
import jax.numpy as jnp
from jax import lax
from jax.experimental import pallas as pl
from jax.experimental.pallas import tpu as pltpu
from jax.experimental.pallas import tpu_sc as plsc

N = 10000
D = 128
NC = 2
NS = 16
LANES = 16
_GRP = 128
_NQ = D // LANES

R_P = 2560
EP = R_P * _GRP
DG_Q = R_P // NS
C_Q = R_P // (NC * NS)

NT = 10112
NP = 10112
ZROWS = NP // NS
WROWS = 624

_sc_mesh = plsc.VectorSubcoreMesh(
    core_axis_name="c", subcore_axis_name="s", num_cores=NC, num_subcores=NS)


def _zero_fill(buf, nrows):
  def zrow(i, carry):
    for qq in range(buf.shape[1] // LANES):
      buf[i, pl.ds(qq * LANES, LANES)] = jnp.zeros((LANES,), jnp.float32)
    return carry
  lax.fori_loop(0, nrows, zrow, 0)


def _zero_table(zb, table, s):
  base = s * ZROWS
  for k in range(4):
    pltpu.sync_copy(zb, table.at[pl.ds(base + k * 128, 128)])
  pltpu.sync_copy(zb.at[pl.ds(0, ZROWS - 512)],
                  table.at[pl.ds(base + 512, ZROWS - 512)])


def _writeout(table, out_hbm, s, out_base):
  wbase = s * WROWS
  pltpu.sync_copy(table.at[pl.ds(wbase, WROWS)],
                  out_hbm.at[pl.ds(out_base + wbase, WROWS)])

  @pl.when(s == NS - 1)
  def _tail():
    tb = NS * WROWS
    pltpu.sync_copy(table.at[pl.ds(tb, N - tb)],
                    out_hbm.at[pl.ds(out_base + tb, N - tb)])


def _deg_body(sd_hbm, degp_hbm, tab_sh, ones_b, ibuf):
  c = lax.axis_index("c")
  s = lax.axis_index("s")

  _zero_fill(ones_b, _GRP)
  _zero_table(ones_b, tab_sh, s)

  def orow(i, carry):
    for qq in range(_NQ):
      ones_b[i, pl.ds(qq * LANES, LANES)] = jnp.ones((LANES,), jnp.float32)
    return carry
  lax.fori_loop(0, _GRP, orow, 0)

  base = c * R_P + s * DG_Q
  pltpu.sync_copy(sd_hbm.at[pl.ds(base, DG_Q)], ibuf)

  plsc.subcore_barrier()

  def grp(j, carry):
    pltpu.sync_copy(ones_b, tab_sh.at[ibuf.at[j]], add=True)
    return carry
  lax.fori_loop(0, DG_Q, grp, 0)

  plsc.subcore_barrier()
  _writeout(tab_sh, degp_hbm, s, c * N)


_deg_call = pl.kernel(
    _deg_body,
    out_type=jax.ShapeDtypeStruct((2 * N, D), jnp.float32),
    mesh=_sc_mesh,
    scratch_types=[
        pltpu.VMEM_SHARED((NP, D), jnp.float32),
        pltpu.VMEM((_GRP, D), jnp.float32),
        pltpu.VMEM((DG_Q, _GRP), jnp.int32),
    ],
)


def _agg_body(xt_hbm, src_hbm, dst_hbm, w_hbm, aggp_hbm,
              agg_sh, sbuf, dbuf, wbuf, rows):
  c = lax.axis_index("c")
  s = lax.axis_index("s")

  _zero_fill(rows, _GRP)
  _zero_table(rows, agg_sh, s)

  base = (c * NS + s) * C_Q
  pltpu.sync_copy(src_hbm.at[pl.ds(base, C_Q)], sbuf)
  pltpu.sync_copy(dst_hbm.at[pl.ds(base, C_Q)], dbuf)
  pltpu.sync_copy(w_hbm.at[pl.ds(base, C_Q)], wbuf)

  plsc.subcore_barrier()

  def grp(j, carry):
    pltpu.sync_copy(xt_hbm.at[sbuf.at[j]], rows)

    def edge16(g, c2):
      wv = wbuf[j, pl.ds(g * LANES, LANES)]
      for i in range(LANES):
        ws = wv[i]
        e = g * LANES + i
        for qq in range(_NQ):
          sl = pl.ds(qq * LANES, LANES)
          rows[e, sl] = rows[e, sl] * ws
      return c2
    lax.fori_loop(0, _GRP // LANES, edge16, 0)
    pltpu.sync_copy(rows, agg_sh.at[dbuf.at[j]], add=True)
    return carry
  lax.fori_loop(0, C_Q, grp, 0)

  plsc.subcore_barrier()
  _writeout(agg_sh, aggp_hbm, s, c * N)


_agg_call = pl.kernel(
    _agg_body,
    out_type=jax.ShapeDtypeStruct((2 * N, D), jnp.float32),
    mesh=_sc_mesh,
    scratch_types=[
        pltpu.VMEM_SHARED((NP, D), jnp.float32),
        pltpu.VMEM((C_Q, _GRP), jnp.int32),
        pltpu.VMEM((C_Q, _GRP), jnp.int32),
        pltpu.VMEM((C_Q, _GRP), jnp.float32),
        pltpu.VMEM((_GRP, D), jnp.float32),
    ],
)


def _prep_body(x_ref, degp_ref, xt_ref):
  outdeg = degp_ref[0:N, 0:1]
  outr = lax.rsqrt(jnp.maximum(outdeg, 1.0))
  xt_ref[0:N, :] = x_ref[...] * outr
  xt_ref[N:NT, :] = jnp.zeros((NT - N, D), jnp.float32)


_prep = pl.pallas_call(
    _prep_body,
    out_shape=jax.ShapeDtypeStruct((NT, D), jnp.float32),
)


def _mid_body(aggp_ref, degp_ref, w_ref, b_ref, out_ref):
  agg = aggp_ref[0:N, :] + aggp_ref[N:2 * N, :]
  indeg = degp_ref[N:2 * N, 0:1]
  inr = lax.rsqrt(jnp.maximum(indeg, 1.0))
  rst = agg * inr
  h = jnp.dot(rst, w_ref[...], preferred_element_type=jnp.float32) + b_ref[...]
  h = jnp.maximum(h, 0.0)
  outdeg = degp_ref[0:N, 0:1]
  outr = lax.rsqrt(jnp.maximum(outdeg, 1.0))
  out_ref[0:N, :] = h * outr
  out_ref[N:NT, :] = jnp.zeros((NT - N, D), jnp.float32)


_mid = pl.pallas_call(
    _mid_body,
    out_shape=jax.ShapeDtypeStruct((NT, D), jnp.float32),
)


def _fin_body(aggp_ref, degp_ref, w_ref, b_ref, out_ref):
  agg = aggp_ref[0:N, :] + aggp_ref[N:2 * N, :]
  indeg = degp_ref[N:2 * N, 0:1]
  inr = lax.rsqrt(jnp.maximum(indeg, 1.0))
  rst = agg * inr
  out_ref[...] = (
      jnp.dot(rst, w_ref[...], preferred_element_type=jnp.float32) + b_ref[...])


_fin = pl.pallas_call(
    _fin_body,
    out_shape=jax.ShapeDtypeStruct((N, D), jnp.float32),
)


@jax.jit
def kernel(node_feats, edge_index, edge_weight, W1, b1, W2, b2):
  E = edge_index.shape[1]
  pad = EP - E
  dummy = N + (jnp.arange(pad, dtype=jnp.int32) % (NP - N))
  src = jnp.concatenate([edge_index[0], dummy]).reshape(R_P, _GRP)
  dst = jnp.concatenate([edge_index[1], dummy]).reshape(R_P, _GRP)
  w2 = jnp.concatenate(
      [edge_weight, jnp.zeros((pad,), jnp.float32)]).reshape(R_P, _GRP)
  degp = _deg_call(jnp.concatenate([src, dst], axis=0))
  xt = _prep(node_feats, degp)
  agg1 = _agg_call(xt, src, dst, w2)
  hh = _mid(agg1, degp, W1, b1.reshape(1, D))
  agg2 = _agg_call(hh, src, dst, w2)
  return _fin(agg2, degp, W2, b2.reshape(1, D))


# speedup vs baseline: 6.7831x; 1.2841x over previous
import jax
import jax.numpy as jnp
from jax import lax
from jax.experimental import pallas as pl
from jax.experimental.pallas import tpu as pltpu
from jax.experimental.pallas import tpu_sc as plsc

N = 10000
D = 128
NC = 2
NS = 16
LANES = 16
_GRP = 128
_NQ = D // LANES
NBUF = 2
CH = 40

R_P = 2560
EP = R_P * _GRP
DG_Q = R_P // NS
C_Q = R_P // (NC * NS)

NT = 10112
NP = 10112
ZROWS = NP // NS
WROWS = 624

_sc_mesh = plsc.VectorSubcoreMesh(
    core_axis_name="c", subcore_axis_name="s", num_cores=NC, num_subcores=NS)


def _zero_fill(buf, nrows):
  def zrow(i, carry):
    for qq in range(buf.shape[1] // LANES):
      buf[i, pl.ds(qq * LANES, LANES)] = jnp.zeros((LANES,), jnp.float32)
    return carry
  lax.fori_loop(0, nrows, zrow, 0)


def _zero_table(zb, table, s):
  base = s * ZROWS
  for k in range(4):
    pltpu.sync_copy(zb, table.at[pl.ds(base + k * 128, 128)])
  pltpu.sync_copy(zb.at[pl.ds(0, ZROWS - 512)],
                  table.at[pl.ds(base + 512, ZROWS - 512)])


def _writeout(table, out_hbm, s, out_base):
  wbase = s * WROWS
  pltpu.sync_copy(table.at[pl.ds(wbase, WROWS)],
                  out_hbm.at[pl.ds(out_base + wbase, WROWS)])

  @pl.when(s == NS - 1)
  def _tail():
    tb = NS * WROWS
    pltpu.sync_copy(table.at[pl.ds(tb, N - tb)],
                    out_hbm.at[pl.ds(out_base + tb, N - tb)])


def _deg_body(sd_hbm, degp_hbm, tab_sh, ones_b, ibuf, dsem):
  c = lax.axis_index("c")
  s = lax.axis_index("s")

  _zero_fill(ones_b, _GRP)
  _zero_table(ones_b, tab_sh, s)

  def orow(i, carry):
    for qq in range(_NQ):
      ones_b[i, pl.ds(qq * LANES, LANES)] = jnp.ones((LANES,), jnp.float32)
    return carry
  lax.fori_loop(0, _GRP, orow, 0)

  base = c * R_P + s * DG_Q
  pltpu.sync_copy(sd_hbm.at[pl.ds(base, DG_Q)], ibuf)

  plsc.subcore_barrier()

  def grp(k, carry):
    descs = []
    for b in range(8):
      j = k * 8 + b
      descs.append(
          pltpu.async_copy(ones_b, tab_sh.at[ibuf.at[j]], dsem, add=True))
    for d in descs:
      d.wait()
    return carry
  lax.fori_loop(0, DG_Q // 8, grp, 0)

  plsc.subcore_barrier()
  _writeout(tab_sh, degp_hbm, s, c * N)


_deg_call = pl.kernel(
    _deg_body,
    out_type=jax.ShapeDtypeStruct((2 * N, D), jnp.float32),
    mesh=_sc_mesh,
    scratch_types=[
        pltpu.VMEM_SHARED((NP, D), jnp.float32),
        pltpu.VMEM((_GRP, D), jnp.float32),
        pltpu.VMEM((DG_Q, _GRP), jnp.int32),
        pltpu.SemaphoreType.DMA,
    ],
)


def _agg_body(xt_hbm, src_hbm, dst_hbm, w_hbm, aggp_hbm,
              agg_sh, sbuf, dbuf, wbuf,
              r0, r1, g0, g1, t0, t1):
  c = lax.axis_index("c")
  s = lax.axis_index("s")
  rows_l = [r0, r1]
  gsem = [g0, g1]
  ssem = [t0, t1]

  _zero_fill(r0, _GRP)
  _zero_table(r0, agg_sh, s)
  plsc.subcore_barrier()

  def scale(rb, wb_j):
    def edge16(g, c2):
      wv = wbuf[wb_j, pl.ds(g * LANES, LANES)]
      for i in range(LANES):
        ws = wv[i]
        e = g * LANES + i
        for qq in range(_NQ):
          sl = pl.ds(qq * LANES, LANES)
          rb[e, sl] = rb[e, sl] * ws
      return c2
    lax.fori_loop(0, _GRP // LANES, edge16, 0)

  for h in range(C_Q // CH):
    base = (c * NS + s) * C_Q + h * CH
    pltpu.sync_copy(src_hbm.at[pl.ds(base, CH)], sbuf)
    pltpu.sync_copy(dst_hbm.at[pl.ds(base, CH)], dbuf)
    pltpu.sync_copy(w_hbm.at[pl.ds(base, CH)], wbuf)

    for b in range(NBUF):
      pltpu.async_copy(xt_hbm.at[sbuf.at[b]], rows_l[b], gsem[b])

    def round_(k, carry):
      descs = []
      for b in range(NBUF):
        j = k * NBUF + b
        pltpu.make_async_copy(xt_hbm.at[sbuf.at[j]], rows_l[b], gsem[b]).wait()
        scale(rows_l[b], j)
        descs.append(
            pltpu.async_copy(rows_l[b], agg_sh.at[dbuf.at[j]], ssem[b],
                             add=True))
      for b in range(NBUF):
        descs[b].wait()
        nj = jnp.minimum(k * NBUF + b + NBUF, CH - 1)
        pltpu.async_copy(xt_hbm.at[sbuf.at[nj]], rows_l[b], gsem[b])
      return carry
    lax.fori_loop(0, CH // NBUF, round_, 0)

    for b in range(NBUF):
      pltpu.make_async_copy(xt_hbm.at[sbuf.at[CH - 1]], rows_l[b],
                            gsem[b]).wait()

  plsc.subcore_barrier()
  _writeout(agg_sh, aggp_hbm, s, c * N)


_agg_call = pl.kernel(
    _agg_body,
    out_type=jax.ShapeDtypeStruct((2 * N, D), jnp.float32),
    mesh=_sc_mesh,
    scratch_types=[
        pltpu.VMEM_SHARED((NP, D), jnp.float32),
        pltpu.VMEM((CH, _GRP), jnp.int32),
        pltpu.VMEM((CH, _GRP), jnp.int32),
        pltpu.VMEM((CH, _GRP), jnp.float32),
        pltpu.VMEM((_GRP, D), jnp.float32),
        pltpu.VMEM((_GRP, D), jnp.float32),
        pltpu.SemaphoreType.DMA,
        pltpu.SemaphoreType.DMA,
        pltpu.SemaphoreType.DMA,
        pltpu.SemaphoreType.DMA,
    ],
)


def _prep_body(x_ref, degp_ref, xt_ref):
  outdeg = degp_ref[0:N, 0:1]
  outr = lax.rsqrt(jnp.maximum(outdeg, 1.0))
  xt_ref[0:N, :] = x_ref[...] * outr
  xt_ref[N:NT, :] = jnp.zeros((NT - N, D), jnp.float32)


_prep = pl.pallas_call(
    _prep_body,
    out_shape=jax.ShapeDtypeStruct((NT, D), jnp.float32),
)


def _mid_body(aggp_ref, degp_ref, w_ref, b_ref, out_ref):
  agg = aggp_ref[0:N, :] + aggp_ref[N:2 * N, :]
  indeg = degp_ref[N:2 * N, 0:1]
  inr = lax.rsqrt(jnp.maximum(indeg, 1.0))
  rst = agg * inr
  h = jnp.dot(rst, w_ref[...], preferred_element_type=jnp.float32) + b_ref[...]
  h = jnp.maximum(h, 0.0)
  outdeg = degp_ref[0:N, 0:1]
  outr = lax.rsqrt(jnp.maximum(outdeg, 1.0))
  out_ref[0:N, :] = h * outr
  out_ref[N:NT, :] = jnp.zeros((NT - N, D), jnp.float32)


_mid = pl.pallas_call(
    _mid_body,
    out_shape=jax.ShapeDtypeStruct((NT, D), jnp.float32),
)


def _fin_body(aggp_ref, degp_ref, w_ref, b_ref, out_ref):
  agg = aggp_ref[0:N, :] + aggp_ref[N:2 * N, :]
  indeg = degp_ref[N:2 * N, 0:1]
  inr = lax.rsqrt(jnp.maximum(indeg, 1.0))
  rst = agg * inr
  out_ref[...] = (
      jnp.dot(rst, w_ref[...], preferred_element_type=jnp.float32) + b_ref[...])


_fin = pl.pallas_call(
    _fin_body,
    out_shape=jax.ShapeDtypeStruct((N, D), jnp.float32),
)


@jax.jit
def kernel(node_feats, edge_index, edge_weight, W1, b1, W2, b2):
  E = edge_index.shape[1]
  pad = EP - E
  dummy = N + (jnp.arange(pad, dtype=jnp.int32) % (NP - N))
  src = jnp.concatenate([edge_index[0], dummy]).reshape(R_P, _GRP)
  dst = jnp.concatenate([edge_index[1], dummy]).reshape(R_P, _GRP)
  w2 = jnp.concatenate(
      [edge_weight, jnp.zeros((pad,), jnp.float32)]).reshape(R_P, _GRP)
  degp = _deg_call(jnp.concatenate([src, dst], axis=0))
  xt = _prep(node_feats, degp)
  agg1 = _agg_call(xt, src, dst, w2)
  hh = _mid(agg1, degp, W1, b1.reshape(1, D))
  agg2 = _agg_call(hh, src, dst, w2)
  return _fin(agg2, degp, W2, b2.reshape(1, D))
